# baseline (speedup 1.0000x reference)
import functools

import jax
import jax.numpy as jnp
from jax import lax
from jax.experimental import pallas as pl
from jax.experimental.pallas import tpu as pltpu
from jax.experimental.pallas import tpu_sc as plsc

NC, NS, L = 2, 16, 16
NW = NC * NS
D = 32
CHUNK = 128


def _worker_id():
    return lax.axis_index("s") * NC + lax.axis_index("c")


def _sc_gather_rows(src, idx):
    n = idx.shape[0]
    per_w = n // NW
    nch = per_w // CHUNK
    mesh = plsc.VectorSubcoreMesh(core_axis_name="c", subcore_axis_name="s")

    @functools.partial(
        pl.kernel,
        out_type=jax.ShapeDtypeStruct((n, D), jnp.float32),
        mesh=mesh,
        scratch_types=[
            pltpu.VMEM((per_w,), jnp.int32),
            pltpu.VMEM((CHUNK, D), jnp.float32),
            pltpu.SemaphoreType.DMA,
        ],
        compiler_params=pltpu.CompilerParams(use_tc_tiling_on_sc=False),
    )
    def k(src_hbm, idx_hbm, out_hbm, iw, rows, sem):
        wid = _worker_id()
        pltpu.sync_copy(idx_hbm.at[pl.ds(wid * per_w, per_w)], iw)

        def chunk(c, _):
            pltpu.async_copy(src_hbm.at[iw.at[pl.ds(c * CHUNK, CHUNK)]], rows, sem).wait()
            pltpu.sync_copy(rows, out_hbm.at[pl.ds(wid * per_w + c * CHUNK, CHUNK)])
            return 0

        lax.fori_loop(0, nch, chunk, 0)

    return k(src, idx)


def _tc_project(emb, W, b2):
    n = emb.shape[0]
    BR = 2048

    def body(e_ref, w_ref, b_ref, p0_ref, p1_ref):
        emb_blk = e_ref[...]
        w = w_ref[...]
        p0_ref[...] = jnp.dot(emb_blk, w[:D, :], preferred_element_type=jnp.float32) + b_ref[...]
        p1_ref[...] = jnp.dot(emb_blk, w[D:, :], preferred_element_type=jnp.float32)

    return pl.pallas_call(
        body,
        grid=(n // BR,),
        in_specs=[
            pl.BlockSpec((BR, D), lambda i: (i, 0)),
            pl.BlockSpec((2 * D, D), lambda i: (0, 0)),
            pl.BlockSpec((1, D), lambda i: (0, 0)),
        ],
        out_specs=(
            pl.BlockSpec((BR, D), lambda i: (i, 0)),
            pl.BlockSpec((BR, D), lambda i: (i, 0)),
        ),
        out_shape=(
            jax.ShapeDtypeStruct((n, D), jnp.float32),
            jax.ShapeDtypeStruct((n, D), jnp.float32),
        ),
    )(emb, W, b2)


def _sc_atoms(p0, p1, a0r, a1r):
    n_rows = a0r.shape[0]
    kc = n_rows // NW
    mesh = plsc.VectorSubcoreMesh(core_axis_name="c", subcore_axis_name="s")

    @functools.partial(
        pl.kernel,
        out_type=jax.ShapeDtypeStruct((n_rows * CHUNK, D), jnp.float32),
        mesh=mesh,
        scratch_types=[
            pltpu.VMEM((kc, CHUNK), jnp.int32),
            pltpu.VMEM((kc, CHUNK), jnp.int32),
            pltpu.VMEM((CHUNK, D), jnp.float32),
            pltpu.VMEM((CHUNK, D), jnp.float32),
            pltpu.VMEM((CHUNK, D), jnp.float32),
            pltpu.SemaphoreType.DMA,
            pltpu.SemaphoreType.DMA,
        ],
        compiler_params=pltpu.CompilerParams(use_tc_tiling_on_sc=False),
    )
    def k(p0_hbm, p1_hbm, a0_hbm, a1_hbm, out_hbm,
          idx0, idx1, r0, r1, ob, sem0, sem1):
        wid = _worker_id()
        pltpu.sync_copy(a0_hbm.at[pl.ds(wid * kc, kc)], idx0)
        pltpu.sync_copy(a1_hbm.at[pl.ds(wid * kc, kc)], idx1)
        for j in range(kc):
            cp0 = pltpu.async_copy(p0_hbm.at[idx0.at[j]], r0, sem0)
            cp1 = pltpu.async_copy(p1_hbm.at[idx1.at[j]], r1, sem1)
            cp0.wait()
            cp1.wait()

            def row(rr, _):
                for h in range(D // L):
                    s = r0[rr, pl.ds(h * L, L)] + r1[rr, pl.ds(h * L, L)]
                    t = jnp.exp(-2.0 * jnp.abs(s))
                    y = (1.0 - t) / (1.0 + t)
                    ob[rr, pl.ds(h * L, L)] = jnp.where(s < 0, -y, y)
                return 0

            lax.fori_loop(0, CHUNK, row, 0)
            pltpu.sync_copy(ob, out_hbm.at[pl.ds((wid * kc + j) * CHUNK, CHUNK)])

    return k(p0, p1, a0r, a1r)


def kernel(X_domains, A_predicates, constant_table, W_atom, b_atom):
    n_atoms = A_predicates.shape[0]
    emb = _sc_gather_rows(constant_table, X_domains)
    p0, p1 = _tc_project(emb, W_atom, b_atom.reshape(1, D))
    a0 = A_predicates[:, 0].reshape(n_atoms // CHUNK, CHUNK)
    a1 = A_predicates[:, 1].reshape(n_atoms // CHUNK, CHUNK)
    return _sc_atoms(p0, p1, a0, a1)


# speedup vs baseline: 1.5060x; 1.5060x over previous
import functools

import jax
import jax.numpy as jnp
from jax import lax
from jax.experimental import pallas as pl
from jax.experimental.pallas import tpu as pltpu
from jax.experimental.pallas import tpu_sc as plsc

NC, NS, L = 2, 16, 16
NW = NC * NS
D = 32
CHUNK = 128


def _worker_id():
    return lax.axis_index("s") * NC + lax.axis_index("c")


def _sc_gather_rows_direct(t, idx):
    n = idx.shape[0]
    per_w = n // NW
    nch = per_w // CHUNK
    mesh = plsc.VectorSubcoreMesh(core_axis_name="c", subcore_axis_name="s")

    @functools.partial(
        pl.kernel,
        out_type=jax.ShapeDtypeStruct((n, D), jnp.float32),
        mesh=mesh,
        scratch_types=[
            pltpu.VMEM((per_w,), jnp.int32),
            pltpu.VMEM((CHUNK, D), jnp.float32),
            pltpu.SemaphoreType.DMA,
        ],
    )
    def k(t_hbm, idx_hbm, out_hbm, iw, rows, sem):
        wid = _worker_id()
        pltpu.sync_copy(idx_hbm.at[pl.ds(wid * per_w, per_w)], iw)

        def chunk(c, _):
            def fire(g, _):
                v = iw[pl.ds(c * CHUNK + g * L, L)]
                for kk in range(L):
                    pltpu.async_copy(t_hbm.at[v[kk]], rows.at[g * L + kk], sem)
                return 0

            lax.fori_loop(0, CHUNK // L, fire, 0)

            def drain(j, _):
                pltpu.make_async_copy(t_hbm.at[0], rows.at[0], sem).wait()
                return 0

            lax.fori_loop(0, CHUNK, drain, 0)
            pltpu.sync_copy(rows, out_hbm.at[pl.ds(wid * per_w + c * CHUNK, CHUNK)])
            return 0

        lax.fori_loop(0, nch, chunk, 0)

    return k(t, idx)


def _tc_project(emb, W, b2):
    n = emb.shape[0]
    BR = 2048

    def body(e_ref, w_ref, b_ref, p0_ref, p1_ref):
        emb_blk = e_ref[...]
        w = w_ref[...]
        p0_ref[...] = jnp.dot(emb_blk, w[:D, :], preferred_element_type=jnp.float32) + b_ref[...]
        p1_ref[...] = jnp.dot(emb_blk, w[D:, :], preferred_element_type=jnp.float32)

    return pl.pallas_call(
        body,
        grid=(n // BR,),
        in_specs=[
            pl.BlockSpec((BR, D), lambda i: (i, 0)),
            pl.BlockSpec((2 * D, D), lambda i: (0, 0)),
            pl.BlockSpec((1, D), lambda i: (0, 0)),
        ],
        out_specs=(
            pl.BlockSpec((BR, D), lambda i: (i, 0)),
            pl.BlockSpec((BR, D), lambda i: (i, 0)),
        ),
        out_shape=(
            jax.ShapeDtypeStruct((n, D), jnp.float32),
            jax.ShapeDtypeStruct((n, D), jnp.float32),
        ),
    )(emb, W, b2)


def _sc_atoms(p0, p1, a0r, a1r):
    n_rows = a0r.shape[0]
    kc = n_rows // NW
    mesh = plsc.VectorSubcoreMesh(core_axis_name="c", subcore_axis_name="s")

    @functools.partial(
        pl.kernel,
        out_type=jax.ShapeDtypeStruct((n_rows * CHUNK, D), jnp.float32),
        mesh=mesh,
        scratch_types=[
            pltpu.VMEM((kc, CHUNK), jnp.int32),
            pltpu.VMEM((kc, CHUNK), jnp.int32),
            pltpu.VMEM((CHUNK, D), jnp.float32),
            pltpu.VMEM((CHUNK, D), jnp.float32),
            pltpu.VMEM((CHUNK, D), jnp.float32),
            pltpu.SemaphoreType.DMA,
            pltpu.SemaphoreType.DMA,
        ],
        compiler_params=pltpu.CompilerParams(use_tc_tiling_on_sc=False),
    )
    def k(p0_hbm, p1_hbm, a0_hbm, a1_hbm, out_hbm,
          idx0, idx1, r0, r1, ob, sem0, sem1):
        wid = _worker_id()
        pltpu.sync_copy(a0_hbm.at[pl.ds(wid * kc, kc)], idx0)
        pltpu.sync_copy(a1_hbm.at[pl.ds(wid * kc, kc)], idx1)
        for j in range(kc):
            cp0 = pltpu.async_copy(p0_hbm.at[idx0.at[j]], r0, sem0)
            cp1 = pltpu.async_copy(p1_hbm.at[idx1.at[j]], r1, sem1)
            cp0.wait()
            cp1.wait()

            def row(rr, _):
                for h in range(D // L):
                    s = r0[rr, pl.ds(h * L, L)] + r1[rr, pl.ds(h * L, L)]
                    t = jnp.exp(-2.0 * jnp.abs(s))
                    y = (1.0 - t) / (1.0 + t)
                    ob[rr, pl.ds(h * L, L)] = jnp.where(s < 0, -y, y)
                return 0

            lax.fori_loop(0, CHUNK, row, 0)
            pltpu.sync_copy(ob, out_hbm.at[pl.ds((wid * kc + j) * CHUNK, CHUNK)])

    return k(p0, p1, a0r, a1r)


def kernel(X_domains, A_predicates, constant_table, W_atom, b_atom):
    n_atoms = A_predicates.shape[0]
    emb = _sc_gather_rows_direct(constant_table, X_domains)
    p0, p1 = _tc_project(emb, W_atom, b_atom.reshape(1, D))
    a0 = A_predicates[:, 0].reshape(n_atoms // CHUNK, CHUNK)
    a1 = A_predicates[:, 1].reshape(n_atoms // CHUNK, CHUNK)
    return _sc_atoms(p0, p1, a0, a1)
